# Initial kernel scaffold; baseline (speedup 1.0000x reference)
import jax
import jax.numpy as jnp
from jax.experimental import pallas as pl


def kernel(x, edge_index, batch, W1, a_src1, a_dst1, b1, W2, a_src2, a_dst2, b2, W3, a_src3, a_dst3, b3, lin_W, lin_b):
    raise NotImplementedError("write your pallas kernel here")


# speedup vs baseline: 18.0048x; 18.0048x over previous
import functools

import jax
import jax.numpy as jnp
from jax import lax
from jax.experimental import pallas as pl
from jax.experimental.pallas import tpu as pltpu
from jax.experimental.pallas import tpu_sc as plsc

NC = 2
NS = 16
LN = 16
CH = 128
BLK = 128
G = 64

_SC_PARAMS = pltpu.CompilerParams(
    needs_layout_passes=False, use_tc_tiling_on_sc=False)


def _proj_and_max(hb, asd_ref, asv_ref, adv_ref, amax_ref, i):
    av = jnp.dot(hb, asd_ref[...], preferred_element_type=jnp.float32)
    asv_ref[...] = av[:, 0:1]
    adv_ref[...] = av[:, 1:2]
    cur = jnp.max(av, axis=0, keepdims=True)

    @pl.when(i == 0)
    def _():
        amax_ref[...] = cur

    @pl.when(i > 0)
    def _():
        amax_ref[...] = jnp.maximum(amax_ref[...], cur)


def _mm_first_body(x_ref, w_ref, asd_ref,
                   h0_ref, h1_ref, asv_ref, adv_ref, amax_ref):
    i = pl.program_id(0)
    hb = jnp.dot(x_ref[...], w_ref[...], preferred_element_type=jnp.float32)
    hh = hb.shape[1] // 2
    h0_ref[...] = hb[:, :hh]
    h1_ref[...] = hb[:, hh:]
    _proj_and_max(hb, asd_ref, asv_ref, adv_ref, amax_ref, i)


def _norm_mm_body(acc_ref, s_ref, b_ref, w_ref, asd_ref,
                  h0_ref, h1_ref, asv_ref, adv_ref, amax_ref):
    i = pl.program_id(0)
    ssum = s_ref[:, 0:1]
    xb = (jnp.concatenate([acc_ref[0], acc_ref[1]], axis=1)
          / (ssum + 1e-16) + b_ref[...])
    xb = jnp.maximum(xb, 0.0)
    hb = jnp.dot(xb, w_ref[...], preferred_element_type=jnp.float32)
    hh = hb.shape[1] // 2
    h0_ref[...] = hb[:, :hh]
    h1_ref[...] = hb[:, hh:]
    _proj_and_max(hb, asd_ref, asv_ref, adv_ref, amax_ref, i)


def _final_body(acc_ref, s_ref, b_ref, batch_ref, lw_ref, lb_ref,
                out_ref, psum_ref, cnt_ref):
    i = pl.program_id(0)
    nb = pl.num_programs(0)

    @pl.when(i == 0)
    def _():
        psum_ref[...] = jnp.zeros_like(psum_ref)
        cnt_ref[...] = jnp.zeros_like(cnt_ref)

    ssum = s_ref[:, 0:1]
    hb = (jnp.concatenate([acc_ref[0], acc_ref[1]], axis=1)
          / (ssum + 1e-16) + b_ref[...])
    bb = batch_ref[0, 0, :].reshape(1, BLK)
    oh = (lax.broadcasted_iota(jnp.int32, (G, BLK), 0)
          == jnp.broadcast_to(bb, (G, BLK))).astype(jnp.float32)
    psum_ref[...] += jnp.dot(oh, hb, preferred_element_type=jnp.float32)
    cnt_ref[...] += jnp.broadcast_to(
        jnp.sum(oh, axis=1, keepdims=True), cnt_ref.shape)

    @pl.when(i == nb - 1)
    def _():
        pooled = psum_ref[...] / jnp.maximum(cnt_ref[...], 1.0)
        out_ref[...] = (jnp.dot(pooled, lw_ref[...],
                                preferred_element_type=jnp.float32)
                        + lb_ref[...])


def _mm_first(x, W, asd, n_pad, d, h):
    nb = n_pad // BLK
    hh = h // 2
    return pl.pallas_call(
        _mm_first_body,
        grid=(nb,),
        in_specs=[
            pl.BlockSpec((BLK, d), lambda i: (i, 0)),
            pl.BlockSpec((d, h), lambda i: (0, 0)),
            pl.BlockSpec((h, 2), lambda i: (0, 0)),
        ],
        out_specs=[
            pl.BlockSpec((BLK, hh), lambda i: (i, 0)),
            pl.BlockSpec((BLK, hh), lambda i: (i, 0)),
            pl.BlockSpec((BLK, 1), lambda i: (i, 0)),
            pl.BlockSpec((BLK, 1), lambda i: (i, 0)),
            pl.BlockSpec((1, 2), lambda i: (0, 0)),
        ],
        out_shape=[
            jax.ShapeDtypeStruct((n_pad, hh), jnp.float32),
            jax.ShapeDtypeStruct((n_pad, hh), jnp.float32),
            jax.ShapeDtypeStruct((n_pad, 1), jnp.float32),
            jax.ShapeDtypeStruct((n_pad, 1), jnp.float32),
            jax.ShapeDtypeStruct((1, 2), jnp.float32),
        ],
    )(x, W, asd)


def _norm_mm(acc, s, b, W, asd, n_pad, h):
    nb = n_pad // BLK
    hh = h // 2
    return pl.pallas_call(
        _norm_mm_body,
        grid=(nb,),
        in_specs=[
            pl.BlockSpec((NC, BLK, hh), lambda i: (0, i, 0)),
            pl.BlockSpec((BLK, LN), lambda i: (i, 0)),
            pl.BlockSpec((1, h), lambda i: (0, 0)),
            pl.BlockSpec((h, h), lambda i: (0, 0)),
            pl.BlockSpec((h, 2), lambda i: (0, 0)),
        ],
        out_specs=[
            pl.BlockSpec((BLK, hh), lambda i: (i, 0)),
            pl.BlockSpec((BLK, hh), lambda i: (i, 0)),
            pl.BlockSpec((BLK, 1), lambda i: (i, 0)),
            pl.BlockSpec((BLK, 1), lambda i: (i, 0)),
            pl.BlockSpec((1, 2), lambda i: (0, 0)),
        ],
        out_shape=[
            jax.ShapeDtypeStruct((n_pad, hh), jnp.float32),
            jax.ShapeDtypeStruct((n_pad, hh), jnp.float32),
            jax.ShapeDtypeStruct((n_pad, 1), jnp.float32),
            jax.ShapeDtypeStruct((n_pad, 1), jnp.float32),
            jax.ShapeDtypeStruct((1, 2), jnp.float32),
        ],
    )(acc, s, b, W, asd)


def _final(acc, s, b, batch3, lwp, lbp, n_pad, h):
    nb = n_pad // BLK
    hh = h // 2
    return pl.pallas_call(
        _final_body,
        grid=(nb,),
        in_specs=[
            pl.BlockSpec((NC, BLK, hh), lambda i: (0, i, 0)),
            pl.BlockSpec((BLK, LN), lambda i: (i, 0)),
            pl.BlockSpec((1, h), lambda i: (0, 0)),
            pl.BlockSpec((1, 1, BLK), lambda i: (i, 0, 0)),
            pl.BlockSpec((h, BLK), lambda i: (0, 0)),
            pl.BlockSpec((1, BLK), lambda i: (0, 0)),
        ],
        out_specs=pl.BlockSpec((G, BLK), lambda i: (0, 0)),
        out_shape=jax.ShapeDtypeStruct((G, BLK), jnp.float32),
        scratch_shapes=[
            pltpu.VMEM((G, BLK), jnp.float32),
            pltpu.VMEM((G, BLK), jnp.float32),
        ],
    )(acc, s, b, batch3, lwp, lbp)


@functools.lru_cache(maxsize=None)
def _make_edge_pass(n_pad, h, cpt, et):
    hh = h // 2
    rps = n_pad // NS
    nzc = rps // 32
    mesh = plsc.VectorSubcoreMesh(core_axis_name="c", subcore_axis_name="s")

    @functools.partial(
        pl.kernel,
        out_type=[
            jax.ShapeDtypeStruct((NC, n_pad, hh), jnp.float32),
            jax.ShapeDtypeStruct((n_pad, LN), jnp.float32),
        ],
        mesh=mesh,
        compiler_params=_SC_PARAMS,
        scratch_types=[
            pltpu.VMEM((n_pad,), jnp.float32),
            pltpu.VMEM((n_pad,), jnp.float32),
            pltpu.VMEM((LN,), jnp.float32),
            pltpu.VMEM((CH,), jnp.int32),
            pltpu.VMEM((CH,), jnp.int32),
            pltpu.VMEM((CH, hh), jnp.float32),
            pltpu.VMEM((CH, LN), jnp.float32),
            pltpu.VMEM((32, hh), jnp.float32),
            pltpu.VMEM((rps, LN), jnp.float32),
            pltpu.VMEM_SHARED((n_pad, hh), jnp.float32),
            pltpu.VMEM_SHARED((n_pad, LN), jnp.float32),
            pltpu.SemaphoreType.DMA,
        ],
    )
    def edge_pass(h0_hbm, h1_hbm, as_hbm, ad_hbm, m_hbm, src_hbm, dst_hbm,
                  zrows_hbm, zs_hbm, acc_out, s_out,
                  as_v, ad_v, m_v, sidx, didx, rows, wrows, zbuf, sbuf,
                  acc_sh, s_sh, sem):
        cid = lax.axis_index("c")
        sid = lax.axis_index("s")
        row0 = sid * rps

        pltpu.sync_copy(as_hbm, as_v)
        pltpu.sync_copy(ad_hbm, ad_v)
        pltpu.sync_copy(m_hbm, m_v)
        pltpu.sync_copy(zrows_hbm, zbuf)
        pltpu.sync_copy(zs_hbm, sbuf)
        pltpu.sync_copy(zs_hbm.at[pl.ds(0, CH)], wrows)

        def zacc(k, carry):
            pltpu.sync_copy(zbuf, acc_sh.at[pl.ds(row0 + k * 32, 32)])
            return carry
        lax.fori_loop(0, nzc, zacc, 0)

        @pl.when(cid == 0)
        def _():
            pltpu.sync_copy(sbuf, s_sh.at[pl.ds(row0, rps)])

        plsc.subcore_barrier()

        mv = m_v[...]
        iota16 = lax.iota(jnp.int32, LN)
        zero16 = jnp.zeros((LN,), jnp.int32)

        def chunk(c, carry):
            base = (sid * cpt + c) * CH
            pltpu.sync_copy(src_hbm.at[pl.ds(base, CH)], sidx)
            pltpu.sync_copy(dst_hbm.at[pl.ds(base, CH)], didx)

            @pl.when(cid == 0)
            def _():
                pltpu.async_copy(h0_hbm.at[sidx], rows, sem).wait()

            @pl.when(cid == 1)
            def _():
                pltpu.async_copy(h1_hbm.at[sidx], rows, sem).wait()

            for j in range(CH // LN):
                s16 = sidx[pl.ds(j * LN, LN)]
                d16 = didx[pl.ds(j * LN, LN)]
                av = plsc.load_gather(as_v, [s16])
                dv = plsc.load_gather(ad_v, [d16])
                t = av + dv
                e = jnp.maximum(t, 0.2 * t)
                w16 = jnp.exp(e - mv)
                gid = base + j * LN + iota16
                w16 = jnp.where(gid < et, w16, 0.0)
                plsc.store_scatter(wrows, [j * LN + iota16, zero16], w16)
                for l in range(LN):
                    r = j * LN + l
                    wb = w16.at[jnp.full((LN,), l, jnp.int32)].get(
                        mode="promise_in_bounds")
                    for k in range(hh // LN):
                        rows[r, pl.ds(k * LN, LN)] = (
                            rows[r, pl.ds(k * LN, LN)] * wb)
            pltpu.sync_copy(rows, acc_sh.at[didx], add=True)

            @pl.when(cid == 0)
            def _():
                pltpu.sync_copy(wrows, s_sh.at[didx], add=True)

            return carry
        lax.fori_loop(0, cpt, chunk, 0)
        plsc.subcore_barrier()

        def wback(k, carry):
            off = row0 + k * 32
            pltpu.sync_copy(acc_sh.at[pl.ds(off, 32)], zbuf)
            pltpu.sync_copy(zbuf, acc_out.at[cid, pl.ds(off, 32)])
            return carry
        lax.fori_loop(0, nzc, wback, 0)

        @pl.when(cid == 0)
        def _():
            pltpu.sync_copy(s_sh.at[pl.ds(row0, rps)], sbuf)
            pltpu.sync_copy(sbuf, s_out.at[pl.ds(row0, rps)])

    return edge_pass


def kernel(x, edge_index, batch, W1, a_src1, a_dst1, b1, W2, a_src2, a_dst2,
           b2, W3, a_src3, a_dst3, b3, lin_W, lin_b):
    n, d = x.shape
    hdim = W1.shape[1]
    c_out = lin_W.shape[1]
    n_pad = -(-n // (NS * 32)) * (NS * 32)
    e = edge_index.shape[1]
    et = e + n
    cpt = -(-et // (NS * CH))
    ept = cpt * NS * CH
    rps = n_pad // NS

    f32 = jnp.float32
    i32 = jnp.int32

    loops = jnp.arange(n, dtype=i32)
    src = jnp.concatenate([edge_index[0].astype(i32), loops,
                           jnp.zeros((ept - et,), i32)])
    dst = jnp.concatenate([edge_index[1].astype(i32), loops,
                           jnp.zeros((ept - et,), i32)])
    x_pad = jnp.pad(x, ((0, n_pad - n), (0, 0)))
    zrows = jnp.zeros((32, hdim // 2), f32)
    zs = jnp.zeros((rps, LN), f32)

    edge_pass = _make_edge_pass(n_pad, hdim, cpt, et)

    def run_edges(h0, h1, asv, adv, amax):
        msum = amax[0, 0] + amax[0, 1]
        m = jnp.maximum(msum, 0.2 * msum)
        mvec = jnp.full((LN,), m, f32)
        return edge_pass(h0, h1, asv.reshape(n_pad), adv.reshape(n_pad),
                         mvec, src, dst, zrows, zs)

    h0, h1, asv, adv, amax = _mm_first(x_pad, W1,
                                       jnp.stack([a_src1, a_dst1], 1),
                                       n_pad, d, hdim)
    acc, sp = run_edges(h0, h1, asv, adv, amax)
    h0, h1, asv, adv, amax = _norm_mm(acc, sp, b1.reshape(1, hdim), W2,
                                      jnp.stack([a_src2, a_dst2], 1),
                                      n_pad, hdim)
    acc, sp = run_edges(h0, h1, asv, adv, amax)
    h0, h1, asv, adv, amax = _norm_mm(acc, sp, b2.reshape(1, hdim), W3,
                                      jnp.stack([a_src3, a_dst3], 1),
                                      n_pad, hdim)
    acc, sp = run_edges(h0, h1, asv, adv, amax)
    batch3 = jnp.pad(batch.astype(i32), (0, n_pad - n),
                     constant_values=G).reshape(n_pad // BLK, 1, BLK)
    lwp = jnp.zeros((hdim, BLK), f32).at[:, :c_out].set(lin_W)
    lbp = jnp.zeros((1, BLK), f32).at[0, :c_out].set(lin_b)
    out = _final(acc, sp, b3.reshape(1, hdim), batch3, lwp, lbp, n_pad, hdim)
    return out[:, :c_out]


# speedup vs baseline: 29.8292x; 1.6567x over previous
import functools

import jax
import jax.numpy as jnp
from jax import lax
from jax.experimental import pallas as pl
from jax.experimental.pallas import tpu as pltpu
from jax.experimental.pallas import tpu_sc as plsc

NC = 2
NS = 16
LN = 16
CH = 128
BLK = 128
G = 64

_SC_PARAMS = pltpu.CompilerParams(
    needs_layout_passes=False, use_tc_tiling_on_sc=False)


def _proj_and_max(hb, asd_ref, asv_ref, adv_ref, amax_ref, i):
    av = jnp.dot(hb, asd_ref[...], preferred_element_type=jnp.float32)
    asv_ref[...] = av[:, 0:1]
    adv_ref[...] = av[:, 1:2]
    cur = jnp.max(av, axis=0, keepdims=True)

    @pl.when(i == 0)
    def _():
        amax_ref[...] = cur

    @pl.when(i > 0)
    def _():
        amax_ref[...] = jnp.maximum(amax_ref[...], cur)


def _mm_first_body(x_ref, w_ref, asd_ref,
                   h0_ref, h1_ref, asv_ref, adv_ref, amax_ref):
    i = pl.program_id(0)
    hb = jnp.dot(x_ref[...], w_ref[...], preferred_element_type=jnp.float32)
    hh = hb.shape[1] // 2
    h0_ref[...] = hb[:, :hh]
    h1_ref[...] = hb[:, hh:]
    _proj_and_max(hb, asd_ref, asv_ref, adv_ref, amax_ref, i)


def _norm_mm_body(acc_ref, s_ref, b_ref, w_ref, asd_ref,
                  h0_ref, h1_ref, asv_ref, adv_ref, amax_ref):
    i = pl.program_id(0)
    ssum = s_ref[:, 0:1]
    xb = (jnp.concatenate([acc_ref[0], acc_ref[1]], axis=1)
          / (ssum + 1e-16) + b_ref[...])
    xb = jnp.maximum(xb, 0.0)
    hb = jnp.dot(xb, w_ref[...], preferred_element_type=jnp.float32)
    hh = hb.shape[1] // 2
    h0_ref[...] = hb[:, :hh]
    h1_ref[...] = hb[:, hh:]
    _proj_and_max(hb, asd_ref, asv_ref, adv_ref, amax_ref, i)


def _final_body(acc_ref, s_ref, b_ref, batch_ref, lw_ref, lb_ref,
                out_ref, psum_ref, cnt_ref):
    i = pl.program_id(0)
    nb = pl.num_programs(0)

    @pl.when(i == 0)
    def _():
        psum_ref[...] = jnp.zeros_like(psum_ref)
        cnt_ref[...] = jnp.zeros_like(cnt_ref)

    ssum = s_ref[:, 0:1]
    hb = (jnp.concatenate([acc_ref[0], acc_ref[1]], axis=1)
          / (ssum + 1e-16) + b_ref[...])
    bb = batch_ref[0, 0, :].reshape(1, BLK)
    oh = (lax.broadcasted_iota(jnp.int32, (G, BLK), 0)
          == jnp.broadcast_to(bb, (G, BLK))).astype(jnp.float32)
    psum_ref[...] += jnp.dot(oh, hb, preferred_element_type=jnp.float32)
    cnt_ref[...] += jnp.broadcast_to(
        jnp.sum(oh, axis=1, keepdims=True), cnt_ref.shape)

    @pl.when(i == nb - 1)
    def _():
        pooled = psum_ref[...] / jnp.maximum(cnt_ref[...], 1.0)
        out_ref[...] = (jnp.dot(pooled, lw_ref[...],
                                preferred_element_type=jnp.float32)
                        + lb_ref[...])


def _mm_first(x, W, asd, n_pad, d, h):
    nb = n_pad // BLK
    hh = h // 2
    return pl.pallas_call(
        _mm_first_body,
        grid=(nb,),
        in_specs=[
            pl.BlockSpec((BLK, d), lambda i: (i, 0)),
            pl.BlockSpec((d, h), lambda i: (0, 0)),
            pl.BlockSpec((h, 2), lambda i: (0, 0)),
        ],
        out_specs=[
            pl.BlockSpec((BLK, hh), lambda i: (i, 0)),
            pl.BlockSpec((BLK, hh), lambda i: (i, 0)),
            pl.BlockSpec((BLK, 1), lambda i: (i, 0)),
            pl.BlockSpec((BLK, 1), lambda i: (i, 0)),
            pl.BlockSpec((1, 2), lambda i: (0, 0)),
        ],
        out_shape=[
            jax.ShapeDtypeStruct((n_pad, hh), jnp.float32),
            jax.ShapeDtypeStruct((n_pad, hh), jnp.float32),
            jax.ShapeDtypeStruct((n_pad, 1), jnp.float32),
            jax.ShapeDtypeStruct((n_pad, 1), jnp.float32),
            jax.ShapeDtypeStruct((1, 2), jnp.float32),
        ],
    )(x, W, asd)


def _norm_mm(acc, s, b, W, asd, n_pad, h):
    nb = n_pad // BLK
    hh = h // 2
    return pl.pallas_call(
        _norm_mm_body,
        grid=(nb,),
        in_specs=[
            pl.BlockSpec((NC, BLK, hh), lambda i: (0, i, 0)),
            pl.BlockSpec((BLK, LN), lambda i: (i, 0)),
            pl.BlockSpec((1, h), lambda i: (0, 0)),
            pl.BlockSpec((h, h), lambda i: (0, 0)),
            pl.BlockSpec((h, 2), lambda i: (0, 0)),
        ],
        out_specs=[
            pl.BlockSpec((BLK, hh), lambda i: (i, 0)),
            pl.BlockSpec((BLK, hh), lambda i: (i, 0)),
            pl.BlockSpec((BLK, 1), lambda i: (i, 0)),
            pl.BlockSpec((BLK, 1), lambda i: (i, 0)),
            pl.BlockSpec((1, 2), lambda i: (0, 0)),
        ],
        out_shape=[
            jax.ShapeDtypeStruct((n_pad, hh), jnp.float32),
            jax.ShapeDtypeStruct((n_pad, hh), jnp.float32),
            jax.ShapeDtypeStruct((n_pad, 1), jnp.float32),
            jax.ShapeDtypeStruct((n_pad, 1), jnp.float32),
            jax.ShapeDtypeStruct((1, 2), jnp.float32),
        ],
    )(acc, s, b, W, asd)


def _final(acc, s, b, batch3, lwp, lbp, n_pad, h):
    nb = n_pad // BLK
    hh = h // 2
    return pl.pallas_call(
        _final_body,
        grid=(nb,),
        in_specs=[
            pl.BlockSpec((NC, BLK, hh), lambda i: (0, i, 0)),
            pl.BlockSpec((BLK, LN), lambda i: (i, 0)),
            pl.BlockSpec((1, h), lambda i: (0, 0)),
            pl.BlockSpec((1, 1, BLK), lambda i: (i, 0, 0)),
            pl.BlockSpec((h, BLK), lambda i: (0, 0)),
            pl.BlockSpec((1, BLK), lambda i: (0, 0)),
        ],
        out_specs=pl.BlockSpec((G, BLK), lambda i: (0, 0)),
        out_shape=jax.ShapeDtypeStruct((G, BLK), jnp.float32),
        scratch_shapes=[
            pltpu.VMEM((G, BLK), jnp.float32),
            pltpu.VMEM((G, BLK), jnp.float32),
        ],
    )(acc, s, b, batch3, lwp, lbp)


@functools.lru_cache(maxsize=None)
def _make_edge_pass(n_pad, h, cpt, et):
    hh = h // 2
    rps = n_pad // NS
    nzc = rps // 32
    mesh = plsc.VectorSubcoreMesh(core_axis_name="c", subcore_axis_name="s")

    @functools.partial(
        pl.kernel,
        out_type=[
            jax.ShapeDtypeStruct((NC, n_pad, hh), jnp.float32),
            jax.ShapeDtypeStruct((n_pad, LN), jnp.float32),
        ],
        mesh=mesh,
        compiler_params=_SC_PARAMS,
        scratch_types=[
            pltpu.VMEM((n_pad,), jnp.float32),
            pltpu.VMEM((n_pad,), jnp.float32),
            pltpu.VMEM((LN,), jnp.float32),
            pltpu.VMEM((CH,), jnp.int32),
            pltpu.VMEM((CH,), jnp.int32),
            pltpu.VMEM((CH,), jnp.int32),
            pltpu.VMEM((CH,), jnp.int32),
            pltpu.VMEM((CH, hh), jnp.float32),
            pltpu.VMEM((CH, hh), jnp.float32),
            pltpu.VMEM((CH, LN), jnp.float32),
            pltpu.VMEM((CH, LN), jnp.float32),
            pltpu.VMEM((CH,), jnp.float32),
            pltpu.VMEM((CH,), jnp.float32),
            pltpu.VMEM((32, hh), jnp.float32),
            pltpu.VMEM((rps, LN), jnp.float32),
            pltpu.VMEM_SHARED((n_pad, hh), jnp.float32),
            pltpu.VMEM_SHARED((n_pad, LN), jnp.float32),
            pltpu.SemaphoreType.DMA,
            pltpu.SemaphoreType.DMA,
            pltpu.SemaphoreType.DMA,
            pltpu.SemaphoreType.DMA,
            pltpu.SemaphoreType.DMA,
            pltpu.SemaphoreType.DMA,
            pltpu.SemaphoreType.DMA,
            pltpu.SemaphoreType.DMA,
            pltpu.SemaphoreType.DMA,
            pltpu.SemaphoreType.DMA,
        ],
    )
    def edge_pass(h0_hbm, h1_hbm, as_hbm, ad_hbm, m_hbm, src_hbm, dst_hbm,
                  zrows_hbm, zs_hbm, acc_out, s_out,
                  as_v, ad_v, m_v, sidx0, sidx1, didx0, didx1, rows0, rows1,
                  wrows0, wrows1, wv0, wv1, zbuf, sbuf, acc_sh, s_sh,
                  ssem0, ssem1, dsem0, dsem1, gsem0, gsem1, asem0, asem1,
                  wsem0, wsem1):
        cid = lax.axis_index("c")
        sid = lax.axis_index("s")
        row0 = sid * rps
        sidxs = (sidx0, sidx1)
        didxs = (didx0, didx1)
        rowss = (rows0, rows1)
        wrowss = (wrows0, wrows1)
        wvs = (wv0, wv1)
        ssems = (ssem0, ssem1)
        dsems = (dsem0, dsem1)
        gsems = (gsem0, gsem1)
        asems = (asem0, asem1)
        wsems = (wsem0, wsem1)

        pltpu.sync_copy(as_hbm, as_v)
        pltpu.sync_copy(ad_hbm, ad_v)
        pltpu.sync_copy(m_hbm, m_v)
        pltpu.sync_copy(zrows_hbm, zbuf)
        pltpu.sync_copy(zs_hbm, sbuf)
        pltpu.sync_copy(zs_hbm.at[pl.ds(0, CH)], wrows0)
        pltpu.sync_copy(zs_hbm.at[pl.ds(0, CH)], wrows1)

        def zacc(k, carry):
            pltpu.sync_copy(zbuf, acc_sh.at[pl.ds(row0 + k * 32, 32)])
            return carry
        lax.fori_loop(0, nzc, zacc, 0)

        @pl.when(cid == 0)
        def _():
            pltpu.sync_copy(sbuf, s_sh.at[pl.ds(row0, rps)])

        plsc.subcore_barrier()

        mv = m_v[...]
        iota16 = lax.iota(jnp.int32, LN)
        zero16 = jnp.zeros((LN,), jnp.int32)

        def issue_idx(c, b):
            base = (sid * cpt + c) * CH
            pltpu.async_copy(src_hbm.at[pl.ds(base, CH)], sidxs[b], ssems[b])
            pltpu.async_copy(dst_hbm.at[pl.ds(base, CH)], didxs[b], dsems[b])

        def wait_idx(b):
            pltpu.make_async_copy(
                src_hbm.at[pl.ds(0, CH)], sidxs[b], ssems[b]).wait()
            pltpu.make_async_copy(
                dst_hbm.at[pl.ds(0, CH)], didxs[b], dsems[b]).wait()

        def issue_gather(b):
            @pl.when(cid == 0)
            def _():
                pltpu.async_copy(h0_hbm.at[sidxs[b]], rowss[b], gsems[b])

            @pl.when(cid == 1)
            def _():
                pltpu.async_copy(h1_hbm.at[sidxs[b]], rowss[b], gsems[b])

        def wait_gather(b):
            pltpu.make_async_copy(
                h0_hbm.at[sidxs[b]], rowss[b], gsems[b]).wait()

        def issue_scat(b):
            pltpu.async_copy(rowss[b], acc_sh.at[didxs[b]], asems[b],
                             add=True)

            @pl.when(cid == 0)
            def _():
                pltpu.async_copy(wrowss[b], s_sh.at[didxs[b]], wsems[b],
                                 add=True)

        def wait_scat(b):
            pltpu.make_async_copy(
                rowss[b], acc_sh.at[didxs[b]], asems[b]).wait()

            @pl.when(cid == 0)
            def _():
                pltpu.make_async_copy(
                    wrowss[b], s_sh.at[didxs[b]], wsems[b]).wait()

        def compute_w(c, b):
            base = (sid * cpt + c) * CH
            for j in range(CH // LN):
                s16 = sidxs[b][pl.ds(j * LN, LN)]
                d16 = didxs[b][pl.ds(j * LN, LN)]
                av = plsc.load_gather(as_v, [s16])
                dv = plsc.load_gather(ad_v, [d16])
                t = av + dv
                e = jnp.maximum(t, 0.2 * t)
                w16 = jnp.exp(e - mv)
                gid = base + j * LN + iota16
                w16 = jnp.where(gid < et, w16, 0.0)
                plsc.store_scatter(wrowss[b], [j * LN + iota16, zero16], w16)
                wvs[b][pl.ds(j * LN, LN)] = w16

        def scale(b):
            for j in range(CH // LN):
                w16 = wvs[b][pl.ds(j * LN, LN)]
                for l in range(LN):
                    r = j * LN + l
                    wb = w16.at[jnp.full((LN,), l, jnp.int32)].get(
                        mode="promise_in_bounds")
                    for k in range(hh // LN):
                        rowss[b][r, pl.ds(k * LN, LN)] = (
                            rowss[b][r, pl.ds(k * LN, LN)] * wb)

        issue_idx(0, 0)
        wait_idx(0)
        issue_gather(0)

        def body2(cc, carry):
            for b in (0, 1):
                i = cc * 2 + b
                nb = 1 - b

                @pl.when(i >= 1)
                def _():
                    wait_scat(nb)

                @pl.when(i < cpt - 1)
                def _():
                    issue_idx(i + 1, nb)

                compute_w(i, b)

                @pl.when(i < cpt - 1)
                def _():
                    wait_idx(nb)
                    issue_gather(nb)

                wait_gather(b)
                scale(b)
                issue_scat(b)
            return carry
        lax.fori_loop(0, cpt // 2, body2, 0)
        wait_scat((cpt - 1) % 2)
        plsc.subcore_barrier()

        def wback(k, carry):
            off = row0 + k * 32
            pltpu.sync_copy(acc_sh.at[pl.ds(off, 32)], zbuf)
            pltpu.sync_copy(zbuf, acc_out.at[cid, pl.ds(off, 32)])
            return carry
        lax.fori_loop(0, nzc, wback, 0)

        @pl.when(cid == 0)
        def _():
            pltpu.sync_copy(s_sh.at[pl.ds(row0, rps)], sbuf)
            pltpu.sync_copy(sbuf, s_out.at[pl.ds(row0, rps)])

    return edge_pass


def kernel(x, edge_index, batch, W1, a_src1, a_dst1, b1, W2, a_src2, a_dst2,
           b2, W3, a_src3, a_dst3, b3, lin_W, lin_b):
    n, d = x.shape
    hdim = W1.shape[1]
    c_out = lin_W.shape[1]
    n_pad = -(-n // (NS * 32)) * (NS * 32)
    e = edge_index.shape[1]
    et = e + n
    cpt = -(-et // (NS * CH))
    cpt += cpt % 2
    ept = cpt * NS * CH
    rps = n_pad // NS

    f32 = jnp.float32
    i32 = jnp.int32

    loops = jnp.arange(n, dtype=i32)
    src = jnp.concatenate([edge_index[0].astype(i32), loops,
                           jnp.zeros((ept - et,), i32)])
    dst = jnp.concatenate([edge_index[1].astype(i32), loops,
                           jnp.zeros((ept - et,), i32)])
    x_pad = jnp.pad(x, ((0, n_pad - n), (0, 0)))
    zrows = jnp.zeros((32, hdim // 2), f32)
    zs = jnp.zeros((rps, LN), f32)

    edge_pass = _make_edge_pass(n_pad, hdim, cpt, et)

    def run_edges(h0, h1, asv, adv, amax):
        msum = amax[0, 0] + amax[0, 1]
        m = jnp.maximum(msum, 0.2 * msum)
        mvec = jnp.full((LN,), m, f32)
        return edge_pass(h0, h1, asv.reshape(n_pad), adv.reshape(n_pad),
                         mvec, src, dst, zrows, zs)

    h0, h1, asv, adv, amax = _mm_first(x_pad, W1,
                                       jnp.stack([a_src1, a_dst1], 1),
                                       n_pad, d, hdim)
    acc, sp = run_edges(h0, h1, asv, adv, amax)
    h0, h1, asv, adv, amax = _norm_mm(acc, sp, b1.reshape(1, hdim), W2,
                                      jnp.stack([a_src2, a_dst2], 1),
                                      n_pad, hdim)
    acc, sp = run_edges(h0, h1, asv, adv, amax)
    h0, h1, asv, adv, amax = _norm_mm(acc, sp, b2.reshape(1, hdim), W3,
                                      jnp.stack([a_src3, a_dst3], 1),
                                      n_pad, hdim)
    acc, sp = run_edges(h0, h1, asv, adv, amax)
    batch3 = jnp.pad(batch.astype(i32), (0, n_pad - n),
                     constant_values=G).reshape(n_pad // BLK, 1, BLK)
    lwp = jnp.zeros((hdim, BLK), f32).at[:, :c_out].set(lin_W)
    lbp = jnp.zeros((1, BLK), f32).at[0, :c_out].set(lin_b)
    out = _final(acc, sp, b3.reshape(1, hdim), batch3, lwp, lbp, n_pad, hdim)
    return out[:, :c_out]
